# baseline (speedup 1.0000x reference)
import functools

import jax
import jax.numpy as jnp
from jax import lax
from jax.experimental import pallas as pl
from jax.experimental.pallas import tpu as pltpu
from jax.experimental.pallas import tpu_sc as plsc

B = 16384
D = 128
NNODES = 1_000_000
NS = 16
L = 16
RANGE = 62504
CHUNK = B // NS
SUB = 128
NSUB = CHUNK // SUB
P1C = 1024
NP1 = B // P1C
RSP = 64


def _body(nodes_hbm, values_hbm, ts_hbm, mem_out_hbm, lu_out_hbm,
          nodes_v, ts_v, win_v, j_all, lu_all, rows_a, rows_b, fix_v,
          sem_j, sem_t, sem_ga, sem_gb, sem_wa, sem_wb,
          win_hbm):
    sid = lax.axis_index("s")
    base = sid * RANGE
    my = sid * CHUNK
    iota = lax.broadcasted_iota(jnp.int32, (L,), 0)
    nxt_idx = jnp.minimum(iota + 1, L - 1)
    notlast = iota < L - 1

    rbufs = (rows_a, rows_b)
    gsems = (sem_ga, sem_gb)
    wsems = (sem_wa, sem_wb)
    gcp = [None, None]
    wcp = [None, None]
    gcp[0] = pltpu.async_copy(values_hbm.at[pl.ds(my, RSP)], rows_a, sem_ga)

    tcp = pltpu.async_copy(ts_hbm, ts_v, sem_t)
    pltpu.sync_copy(nodes_hbm, nodes_v)

    for k in range(NP1):
        koff = k * P1C

        def p1(t, acc, _koff=koff):
            n = nodes_v[pl.ds(_koff + t * L, L)]
            rel = n - base
            m = (rel >= 0) & (rel < RANGE)
            jv = _koff + t * L + iota
            plsc.store_scatter(win_v, [rel], jv, mask=m)
            r = plsc.load_gather(win_v, [rel], mask=m)
            bad = m & (r != jv)
            return acc + plsc.all_reduce_population_count(bad)

        acc = lax.fori_loop(0, P1C // L, p1, jnp.zeros((L,), jnp.int32))

        @pl.when(jnp.any(acc > 0))
        def _(_koff=koff):
            def rp(t, carry):
                n = nodes_v[pl.ds(_koff + t * L, L)]
                key = n * L + iota
                skey, _ = plsc.sort_key_val(key, key)
                n_s = skey >> 4
                j_s = _koff + t * L + (skey & (L - 1))
                nxt = n_s.at[nxt_idx].get(mode="promise_in_bounds")
                loser = (n_s == nxt) & notlast
                rel_s = n_s - base
                m2 = ((rel_s >= 0) & (rel_s < RANGE)
                      & jnp.logical_not(loser))
                plsc.store_scatter(win_v, [rel_s], j_s, mask=m2)
                return carry

            lax.fori_loop(0, P1C // L, rp, 0)

        b = k % 2
        gcp[b].wait()
        wcp[b] = pltpu.async_copy(
            rbufs[b], mem_out_hbm.at[pl.ds(my + k * RSP, RSP)], wsems[b])
        if k + 1 < NP1:
            nb = (k + 1) % 2
            if k >= 1:
                wcp[nb].wait()
            gcp[nb] = pltpu.async_copy(
                values_hbm.at[pl.ds(my + (k + 1) * RSP, RSP)],
                rbufs[nb], gsems[nb])

    wcp[0].wait()
    wcp[1].wait()

    pltpu.sync_copy(win_v, win_hbm.at[pl.ds(base, RANGE)])
    plsc.subcore_barrier()

    jcps = [pltpu.async_copy(win_hbm.at[nodes_v.at[pl.ds(my + c * SUB, SUB)]],
                             j_all.at[pl.ds(c * SUB, SUB)], sem_j)
            for c in range(NSUB)]
    for c in jcps:
        c.wait()
    tcp.wait()

    def fix(g, carry):
        jv = j_all[pl.ds(g * L, L)]
        lu_all[pl.ds(g * L, L)] = plsc.load_gather(ts_v, [jv])
        bad = jv != my + g * L + iota

        @pl.when(jnp.any(bad))
        def _():
            pltpu.sync_copy(values_hbm.at[j_all.at[pl.ds(g * L, L)]], fix_v)
            pltpu.sync_copy(fix_v, mem_out_hbm.at[pl.ds(my + g * L, L)])

        return carry

    lax.fori_loop(0, CHUNK // L, fix, 0)

    pltpu.sync_copy(lu_all, lu_out_hbm.at[pl.ds(my, CHUNK)])


_dedup_gather = functools.partial(
    pl.kernel,
    out_type=(
        jax.ShapeDtypeStruct((B, D), jnp.float32),
        jax.ShapeDtypeStruct((B,), jnp.float32),
    ),
    mesh=plsc.VectorSubcoreMesh(core_axis_name="c", subcore_axis_name="s",
                                num_cores=1),
    compiler_params=pltpu.CompilerParams(needs_layout_passes=False),
    scratch_types=[
        pltpu.VMEM((B,), jnp.int32),
        pltpu.VMEM((B,), jnp.float32),
        pltpu.VMEM((RANGE,), jnp.int32),
        pltpu.VMEM((CHUNK,), jnp.int32),
        pltpu.VMEM((CHUNK,), jnp.float32),
        pltpu.VMEM((RSP, D), jnp.float32),
        pltpu.VMEM((RSP, D), jnp.float32),
        pltpu.VMEM((L, D), jnp.float32),
        pltpu.SemaphoreType.DMA,
        pltpu.SemaphoreType.DMA,
        pltpu.SemaphoreType.DMA,
        pltpu.SemaphoreType.DMA,
        pltpu.SemaphoreType.DMA,
        pltpu.SemaphoreType.DMA,
        pltpu.HBM((NS * RANGE,), jnp.int32),
    ],
)(_body)


def kernel(memory, last_update, nodes, values, ts):
    mem_out, lu_out = _dedup_gather(nodes, values, ts)
    return mem_out, lu_out


# speedup vs baseline: 1.1040x; 1.1040x over previous
import functools

import jax
import jax.numpy as jnp
from jax import lax
from jax.experimental import pallas as pl
from jax.experimental.pallas import tpu as pltpu
from jax.experimental.pallas import tpu_sc as plsc

B = 16384
D = 128
NNODES = 1_000_000
NS = 16
L = 16
RANGE = 62504
CHUNK = B // NS
SUB = 128
NSUB = CHUNK // SUB
P1C = 1024
NP1 = B // P1C
RSP = 128


def _body(nodes_hbm, values_hbm, ts_hbm, mem_out_hbm, lu_out_hbm,
          nodes_v, ts_v, win_v, j_all, lu_all, rows_a, rows_b, fix_v,
          sem_j, sem_t, sem_ga, sem_gb, sem_wa, sem_wb,
          win_hbm):
    sid = lax.axis_index("s")
    base = sid * RANGE
    my = sid * CHUNK
    iota = lax.broadcasted_iota(jnp.int32, (L,), 0)
    nxt_idx = jnp.minimum(iota + 1, L - 1)
    notlast = iota < L - 1

    rbufs = (rows_a, rows_b)
    gsems = (sem_ga, sem_gb)
    wsems = (sem_wa, sem_wb)
    gcp = [None, None]
    wcp = [None, None]
    gcp[0] = pltpu.async_copy(values_hbm.at[pl.ds(my, RSP)], rows_a, sem_ga)

    tcp = pltpu.async_copy(ts_hbm, ts_v, sem_t)
    pltpu.sync_copy(nodes_hbm, nodes_v)

    with jax.named_scope("p1_scan"):
        for k in range(NP1):
            koff = k * P1C

            def p1(t, acc, _koff=koff):
                n = nodes_v[pl.ds(_koff + t * L, L)]
                rel = n - base
                m = (rel >= 0) & (rel < RANGE)
                jv = _koff + t * L + iota
                plsc.store_scatter(win_v, [rel], jv, mask=m)
                r = plsc.load_gather(win_v, [rel], mask=m)
                bad = m & (r != jv)
                return acc + plsc.all_reduce_population_count(bad)

            acc = lax.fori_loop(0, P1C // L, p1, jnp.zeros((L,), jnp.int32))

            @pl.when(jnp.any(acc > 0))
            def _(_koff=koff):
                def rp(t, carry):
                    n = nodes_v[pl.ds(_koff + t * L, L)]
                    key = n * L + iota
                    skey, _ = plsc.sort_key_val(key, key)
                    n_s = skey >> 4
                    j_s = _koff + t * L + (skey & (L - 1))
                    nxt = n_s.at[nxt_idx].get(mode="promise_in_bounds")
                    loser = (n_s == nxt) & notlast
                    rel_s = n_s - base
                    m2 = ((rel_s >= 0) & (rel_s < RANGE)
                          & jnp.logical_not(loser))
                    plsc.store_scatter(win_v, [rel_s], j_s, mask=m2)
                    return carry

                lax.fori_loop(0, P1C // L, rp, 0)

            if k % 2 == 0:
                s = k // 2
                b = s % 2
                gcp[b].wait()
                wcp[b] = pltpu.async_copy(
                    rbufs[b], mem_out_hbm.at[pl.ds(my + s * RSP, RSP)],
                    wsems[b])
                if s + 1 < NP1 // 2:
                    nb = (s + 1) % 2
                    if s >= 1:
                        wcp[nb].wait()
                    gcp[nb] = pltpu.async_copy(
                        values_hbm.at[pl.ds(my + (s + 1) * RSP, RSP)],
                        rbufs[nb], gsems[nb])

        wcp[0].wait()
        wcp[1].wait()

    with jax.named_scope("p1_publish"):
        pltpu.sync_copy(win_v, win_hbm.at[pl.ds(base, RANGE)])
        plsc.subcore_barrier()

    with jax.named_scope("p2_jgather"):
        jcps = [pltpu.async_copy(
                    win_hbm.at[nodes_v.at[pl.ds(my + c * SUB, SUB)]],
                    j_all.at[pl.ds(c * SUB, SUB)], sem_j)
                for c in range(NSUB)]
        for c in jcps:
            c.wait()
        tcp.wait()

    with jax.named_scope("p2_fixup"):
        def fix(g, carry):
            jv = j_all[pl.ds(g * L, L)]
            lu_all[pl.ds(g * L, L)] = plsc.load_gather(ts_v, [jv])
            bad = jv != my + g * L + iota

            @pl.when(jnp.any(bad))
            def _():
                fixb = rows_a.at[pl.ds(0, L)]
                pltpu.sync_copy(values_hbm.at[j_all.at[pl.ds(g * L, L)]],
                                fixb)
                pltpu.sync_copy(fixb, mem_out_hbm.at[pl.ds(my + g * L, L)])

            return carry

        lax.fori_loop(0, CHUNK // L, fix, 0)

        pltpu.sync_copy(lu_all, lu_out_hbm.at[pl.ds(my, CHUNK)])


_dedup_gather = functools.partial(
    pl.kernel,
    out_type=(
        jax.ShapeDtypeStruct((B, D), jnp.float32),
        jax.ShapeDtypeStruct((B,), jnp.float32),
    ),
    mesh=plsc.VectorSubcoreMesh(core_axis_name="c", subcore_axis_name="s",
                                num_cores=1),
    compiler_params=pltpu.CompilerParams(needs_layout_passes=False),
    scratch_types=[
        pltpu.VMEM((B,), jnp.int32),
        pltpu.VMEM((B,), jnp.float32),
        pltpu.VMEM((RANGE,), jnp.int32),
        pltpu.VMEM((CHUNK,), jnp.int32),
        pltpu.VMEM((CHUNK,), jnp.float32),
        pltpu.VMEM((RSP, D), jnp.float32),
        pltpu.VMEM((RSP, D), jnp.float32),
        pltpu.VMEM((L, D), jnp.float32),
        pltpu.SemaphoreType.DMA,
        pltpu.SemaphoreType.DMA,
        pltpu.SemaphoreType.DMA,
        pltpu.SemaphoreType.DMA,
        pltpu.SemaphoreType.DMA,
        pltpu.SemaphoreType.DMA,
        pltpu.HBM((NS * RANGE,), jnp.int32),
    ],
)(_body)


def kernel(memory, last_update, nodes, values, ts):
    mem_out, lu_out = _dedup_gather(nodes, values, ts)
    return mem_out, lu_out


# speedup vs baseline: 1.2709x; 1.1512x over previous
import functools

import jax
import jax.numpy as jnp
from jax import lax
from jax.experimental import pallas as pl
from jax.experimental.pallas import tpu as pltpu
from jax.experimental.pallas import tpu_sc as plsc

B = 16384
D = 128
NNODES = 1_000_000
NS = 16
L = 16
RANGE = 62504
CHUNK = B // NS
SUB = 128
NSUB = CHUNK // SUB
P1C = 1024
NP1 = B // P1C
RSP = 64
NRB = 3
FTILE = 64
NFT = 3
FCAP = NFT * FTILE


def _body(nodes_hbm, values_hbm, ts_hbm, mem_out_hbm, lu_out_hbm,
          nodes_v, ts_v, win_v, j_all, lu_all, r0, r1, r2,
          src_l, dst_l,
          sem_j, sem_t, sg0, sg1, sg2, sw0, sw1, sw2,
          win_hbm):
    sid = lax.axis_index("s")
    base = sid * RANGE
    my = sid * CHUNK
    iota = lax.broadcasted_iota(jnp.int32, (L,), 0)
    nxt_idx = jnp.minimum(iota + 1, L - 1)
    notlast = iota < L - 1

    rbufs = (r0, r1, r2)
    gsems = (sg0, sg1, sg2)
    wsems = (sw0, sw1, sw2)
    gcp = [None] * NP1
    wcp = [None] * NP1
    for s in range(NRB - 1):
        gcp[s] = pltpu.async_copy(values_hbm.at[pl.ds(my + s * RSP, RSP)],
                                  rbufs[s], gsems[s])

    tcp = pltpu.async_copy(ts_hbm, ts_v, sem_t)
    pltpu.sync_copy(nodes_hbm, nodes_v)

    with jax.named_scope("p1_scan"):
        for k in range(NP1):
            koff = k * P1C

            def p1(t, acc, _koff=koff):
                n = nodes_v[pl.ds(_koff + t * L, L)]
                rel = n - base
                m = (rel >= 0) & (rel < RANGE)
                jv = _koff + t * L + iota
                plsc.store_scatter(win_v, [rel], jv, mask=m)
                r = plsc.load_gather(win_v, [rel], mask=m)
                bad = m & (r != jv)
                return acc + plsc.all_reduce_population_count(bad)

            acc = lax.fori_loop(0, P1C // L, p1, jnp.zeros((L,), jnp.int32))

            @pl.when(jnp.any(acc > 0))
            def _(_koff=koff):
                def rp(t, carry):
                    n = nodes_v[pl.ds(_koff + t * L, L)]
                    key = n * L + iota
                    skey, _ = plsc.sort_key_val(key, key)
                    n_s = skey >> 4
                    j_s = _koff + t * L + (skey & (L - 1))
                    nxt = n_s.at[nxt_idx].get(mode="promise_in_bounds")
                    loser = (n_s == nxt) & notlast
                    rel_s = n_s - base
                    m2 = ((rel_s >= 0) & (rel_s < RANGE)
                          & jnp.logical_not(loser))
                    plsc.store_scatter(win_v, [rel_s], j_s, mask=m2)
                    return carry

                lax.fori_loop(0, P1C // L, rp, 0)

            s = k
            b = s % NRB
            gcp[s].wait()
            wcp[s] = pltpu.async_copy(
                rbufs[b], mem_out_hbm.at[pl.ds(my + s * RSP, RSP)],
                wsems[b])
            nxt = s + NRB - 1
            if nxt < NP1:
                if nxt - NRB >= 0:
                    wcp[nxt - NRB].wait()
                gcp[nxt] = pltpu.async_copy(
                    values_hbm.at[pl.ds(my + nxt * RSP, RSP)],
                    rbufs[nxt % NRB], gsems[nxt % NRB])

        for s in range(NP1):
            if wcp[s] is not None and s >= NP1 - NRB:
                wcp[s].wait()

    with jax.named_scope("p1_publish"):
        pltpu.sync_copy(win_v, win_hbm.at[pl.ds(base, RANGE)])
        plsc.subcore_barrier()

    with jax.named_scope("p2_jgather"):
        jcps = [pltpu.async_copy(
                    win_hbm.at[nodes_v.at[pl.ds(my + c * SUB, SUB)]],
                    j_all.at[pl.ds(c * SUB, SUB)], sem_j)
                for c in range(NSUB)]
        for c in jcps:
            c.wait()
        tcp.wait()

    with jax.named_scope("p2_fixup"):
        j0 = j_all[pl.ds(0, L)]
        j0 = j0.at[jnp.zeros((L,), jnp.int32)].get(mode="promise_in_bounds")
        myv = my + iota * 0

        def pre(q, carry):
            for c in range(FTILE // L):
                src_l[q, pl.ds(c * L, L)] = j0
                dst_l[q, pl.ds(c * L, L)] = myv
            return carry

        lax.fori_loop(0, NFT, pre, 0)

        def fix(g, cnt):
            jv = j_all[pl.ds(g * L, L)]
            lu_all[pl.ds(g * L, L)] = plsc.load_gather(ts_v, [jv])
            expect = my + g * L + iota
            bad = jv != expect
            pos = cnt + plsc.cumsum(bad.astype(jnp.int32)) - 1
            row = pos >> 6
            col = pos & (FTILE - 1)
            mlist = bad & (pos < FCAP)
            plsc.store_scatter(src_l, [row, col], jv, mask=mlist)
            plsc.store_scatter(dst_l, [row, col], expect, mask=mlist)
            return cnt + plsc.all_reduce_population_count(bad)

        cnt = lax.fori_loop(0, CHUNK // L, fix,
                            jnp.zeros((L,), jnp.int32))
        cnt_s = jnp.max(cnt)
        ntiles = (jnp.minimum(cnt_s, FCAP) + FTILE - 1) // FTILE

        fixb = r0

        def tile(q, carry):
            pltpu.sync_copy(values_hbm.at[src_l.at[q]], fixb)
            pltpu.sync_copy(fixb, mem_out_hbm.at[dst_l.at[q]])
            return carry

        lax.fori_loop(0, ntiles, tile, 0)

        @pl.when(cnt_s > FCAP)
        def _():
            fixs = r1.at[pl.ds(0, L)]

            def slowfix(g, carry):
                jv = j_all[pl.ds(g * L, L)]
                bad2 = jv != my + g * L + iota

                @pl.when(jnp.any(bad2))
                def _():
                    pltpu.sync_copy(
                        values_hbm.at[j_all.at[pl.ds(g * L, L)]], fixs)
                    pltpu.sync_copy(
                        fixs, mem_out_hbm.at[pl.ds(my + g * L, L)])

                return carry

            lax.fori_loop(0, CHUNK // L, slowfix, 0)

        pltpu.sync_copy(lu_all, lu_out_hbm.at[pl.ds(my, CHUNK)])


_dedup_gather = functools.partial(
    pl.kernel,
    out_type=(
        jax.ShapeDtypeStruct((B, D), jnp.float32),
        jax.ShapeDtypeStruct((B,), jnp.float32),
    ),
    mesh=plsc.VectorSubcoreMesh(core_axis_name="c", subcore_axis_name="s",
                                num_cores=1),
    compiler_params=pltpu.CompilerParams(needs_layout_passes=False),
    scratch_types=[
        pltpu.VMEM((B,), jnp.int32),
        pltpu.VMEM((B,), jnp.float32),
        pltpu.VMEM((RANGE,), jnp.int32),
        pltpu.VMEM((CHUNK,), jnp.int32),
        pltpu.VMEM((CHUNK,), jnp.float32),
        pltpu.VMEM((RSP, D), jnp.float32),
        pltpu.VMEM((RSP, D), jnp.float32),
        pltpu.VMEM((RSP, D), jnp.float32),
        pltpu.VMEM((NFT, FTILE), jnp.int32),
        pltpu.VMEM((NFT, FTILE), jnp.int32),
        pltpu.SemaphoreType.DMA,
        pltpu.SemaphoreType.DMA,
        pltpu.SemaphoreType.DMA,
        pltpu.SemaphoreType.DMA,
        pltpu.SemaphoreType.DMA,
        pltpu.SemaphoreType.DMA,
        pltpu.SemaphoreType.DMA,
        pltpu.SemaphoreType.DMA,
        pltpu.HBM((NS * RANGE,), jnp.int32),
    ],
)(_body)


def kernel(memory, last_update, nodes, values, ts):
    mem_out, lu_out = _dedup_gather(nodes, values, ts)
    return mem_out, lu_out
